# baseline (speedup 1.0000x reference)
import functools

import jax
import jax.numpy as jnp
from jax import lax
from jax.experimental import pallas as pl
from jax.experimental.pallas import tpu as pltpu
from jax.experimental.pallas import tpu_sc as plsc

N = 10000
NP = 10240
E = 160000
D = 64
K = 4
BS = 8
NT = 16
NCORE = 2
ET = E // NT
C = 4
RC = NP // C
RT = RC // NT
G = 48


WROWS = 64
NWIN = NP // WROWS
WPT = NWIN // (NT * NCORE)
WGMAX = 8192
G = 16
EP = E + WGMAX
WSL = 176


def _vgather(x, idx):
    dn = lax.GatherDimensionNumbers(offset_dims=(), collapsed_slice_dims=(0,),
                                    start_index_map=(0,))
    return lax.gather(x, idx[:, None], dn, (1,),
                      mode=lax.GatherScatterMode.PROMISE_IN_BOUNDS)


def _make_spmm(W):
    WV = W // 16
    mesh = plsc.VectorSubcoreMesh(core_axis_name="c", subcore_axis_name="s",
                                  num_cores=NCORE, num_subcores=NT)

    @functools.partial(
        pl.kernel,
        out_type=jax.ShapeDtypeStruct((NP, W), jnp.float32),
        mesh=mesh,
        scratch_types=[
            pltpu.VMEM((WGMAX,), jnp.int32),
            pltpu.VMEM((WGMAX,), jnp.int32),
            pltpu.VMEM((WGMAX,), jnp.float32),
            pltpu.VMEM((WSL,), jnp.int32),
            pltpu.VMEM((WROWS + 8, W), jnp.float32),
            pltpu.VMEM((2, G, W), jnp.float32),
            pltpu.VMEM((2, G), jnp.int32),
            pltpu.VMEM((32,), jnp.int32),
            pltpu.SemaphoreType.DMA,
        ],
    )
    def spmm(x_hbm, rows_hbm, cols_hbm, vals_hbm, ws_hbm, y_hbm,
             rows_v, cols_v, vals_v, wsv, acc, gbuf, gidx, dbuf, gsem):
        cid = lax.axis_index("c")
        sid = lax.axis_index("s")
        wid = cid * NT + sid
        w0 = wid * WPT

        pltpu.sync_copy(ws_hbm, wsv)
        vstart = wsv[pl.ds(w0, 16)]
        e_start = vstart[0]
        e_end = vstart[WPT]
        astart = (e_start // 8) * 8
        e_end = jnp.minimum(e_end, astart + WGMAX)
        pltpu.sync_copy(rows_hbm.at[pl.ds(astart, WGMAX)], rows_v)
        pltpu.sync_copy(cols_hbm.at[pl.ds(astart, WGMAX)], cols_v)
        pltpu.sync_copy(vals_hbm.at[pl.ds(astart, WGMAX)], vals_v)

        zf = jnp.zeros((16,), jnp.float32)
        zi = jnp.zeros((16,), jnp.int32)
        iota = lax.iota(jnp.int32, 16)
        dbuf[pl.ds(16, 16)] = zi

        def zacc(r, _):
            for w in range(WV):
                acc[r, pl.ds(w * 16, 16)] = zf
            return 0
        lax.fori_loop(0, WROWS + 1, zacc, 0)

        for j in range(WPT):
            wlo = (w0 + j) * WROWS
            es = jnp.minimum(vstart[j], e_end)
            ee = jnp.minimum(vstart[j + 1], e_end)
            cnt = ee - es
            base0 = es - astart
            nb = (cnt + (G - 1)) // G

            def fill_gidx(k, buf):
                cc = cols_v[pl.ds(base0 + k * G, 16)]
                keep = (jnp.full((16,), k * G, jnp.int32) + iota) < jnp.full(
                    (16,), cnt, jnp.int32)
                gidx[buf, pl.ds(0, 16)] = jnp.where(keep, cc, zi)
                pltpu.make_async_copy(
                    x_hbm.at[gidx.at[buf]], gbuf.at[buf], gsem).start()

            @pl.when(nb > 0)
            def _():
                fill_gidx(0, 0)

            def blk(k, _):
                buf = lax.rem(k, 2)
                pltpu.make_async_copy(
                    x_hbm.at[gidx.at[buf]], gbuf.at[buf], gsem).wait()
                @pl.when(k + 1 < nb)
                def _():
                    fill_gidx(k + 1, 1 - buf)
                rv = rows_v[pl.ds(base0 + k * G, 16)]
                vv = vals_v[pl.ds(base0 + k * G, 16)]
                keep = (jnp.full((16,), k * G, jnp.int32) + iota) < jnp.full(
                    (16,), cnt, jnp.int32)
                dstv = jnp.where(keep, rv - jnp.full((16,), wlo, jnp.int32),
                                 jnp.full((16,), WROWS, jnp.int32))
                vk = jnp.where(keep, vv, zf)
                dbuf[pl.ds(0, 16)] = dstv

                def lane4(q, _):
                    for li in range(4):
                        l = q * 4 + li
                        d = dbuf[pl.ds(l, 16)][0]
                        bv = _vgather(vk, jnp.full((16,), l, jnp.int32))
                        for w in range(WV):
                            acc[d, pl.ds(w * 16, 16)] = (
                                acc[d, pl.ds(w * 16, 16)]
                                + gbuf[buf, l, pl.ds(w * 16, 16)] * bv)
                    return 0
                lax.fori_loop(0, G // 4, lane4, 0)
                return 0
            lax.fori_loop(0, nb, blk, 0)

            pltpu.sync_copy(acc.at[pl.ds(0, WROWS)],
                            y_hbm.at[pl.ds(wlo, WROWS)])
            lax.fori_loop(0, WROWS + 1, zacc, 0)

    return spmm


_spmm_640 = _make_spmm(640)
_spmm_512 = _make_spmm(512)


_BN = 400
_BNP = 1024


def _scores_body(hx_ref, r_ref, aw_ref, s_ref, c_ref):
    i = pl.program_id(0)
    awb = aw_ref[...]
    ps = jnp.sum(hx_ref[...] * awb[None], axis=1)
    pc = jnp.sum(r_ref[...] * awb[None], axis=1)

    @pl.when(i == 0)
    def _():
        s_ref[...] = jnp.zeros_like(s_ref)
        c_ref[...] = jnp.zeros_like(c_ref)
    s_ref[...] += ps
    c_ref[...] += pc


def _scores(hx_l, r_l, aw_l):
    hx2 = hx_l.reshape(32, N, 64)
    s_part, c_part = pl.pallas_call(
        _scores_body,
        grid=(N // _BN,),
        in_specs=[
            pl.BlockSpec((32, _BN, 64), lambda i: (0, i, 0)),
            pl.BlockSpec((4, _BN, 64), lambda i: (0, i, 0)),
            pl.BlockSpec((_BN, 64), lambda i: (i, 0)),
        ],
        out_specs=[
            pl.BlockSpec((32, 64), lambda i: (0, 0)),
            pl.BlockSpec((4, 64), lambda i: (0, 0)),
        ],
        out_shape=[
            jax.ShapeDtypeStruct((32, 64), jnp.float32),
            jax.ShapeDtypeStruct((4, 64), jnp.float32),
        ],
    )(hx2, r_l, aw_l)
    return s_part.sum(axis=1).reshape(8, 4), c_part.sum(axis=1)


def _proj0_body(x0_ref, y1_ref, bd02_ref, bd1_ref, bd2_ref, u_ref, pa_ref):
    y1 = y1_ref[...]
    u_ref[...] = jnp.dot(y1, bd2_ref[...], preferred_element_type=jnp.float32)
    pa_ref[...] = (
        jnp.dot(x0_ref[...], bd02_ref[...], preferred_element_type=jnp.float32)
        + jnp.dot(y1, bd1_ref[...], preferred_element_type=jnp.float32))


def _proj0(x0p, y1, bd02, bd1, bd2):
    return pl.pallas_call(
        _proj0_body,
        grid=(NP // _BNP,),
        in_specs=[
            pl.BlockSpec((_BNP, 640), lambda i: (i, 0)),
            pl.BlockSpec((_BNP, 640), lambda i: (i, 0)),
            pl.BlockSpec((640, 512), lambda i: (0, 0)),
            pl.BlockSpec((640, 512), lambda i: (0, 0)),
            pl.BlockSpec((640, 512), lambda i: (0, 0)),
        ],
        out_specs=[
            pl.BlockSpec((_BNP, 512), lambda i: (i, 0)),
            pl.BlockSpec((_BNP, 512), lambda i: (i, 0)),
        ],
        out_shape=[
            jax.ShapeDtypeStruct((NP, 512), jnp.float32),
            jax.ShapeDtypeStruct((NP, 512), jnp.float32),
        ],
    )(x0p, y1, bd02, bd1, bd2)


def _proj1_body(x0a_ref, x0b_ref, ya_ref, yb_ref,
                bd02_ref, bd1_ref, bd2_ref, u_ref, pa_ref):
    bd02, bd1, bd2 = bd02_ref[...], bd1_ref[...], bd2_ref[...]
    ya = ya_ref[...]
    yb = yb_ref[...]
    ua = jnp.dot(ya, bd2, preferred_element_type=jnp.float32)
    ub = jnp.dot(yb, bd2, preferred_element_type=jnp.float32)
    u_ref[...] = jnp.concatenate([ua, ub], axis=1)
    pa = (jnp.dot(x0a_ref[...], bd02, preferred_element_type=jnp.float32)
          + jnp.dot(ya, bd1, preferred_element_type=jnp.float32))
    pb = (jnp.dot(x0b_ref[...], bd02, preferred_element_type=jnp.float32)
          + jnp.dot(yb, bd1, preferred_element_type=jnp.float32))
    pa_ref[...] = jnp.concatenate([pa, pb], axis=1)


def _proj1(x1a, x1b, ya, yb, bd02, bd1, bd2):
    return pl.pallas_call(
        _proj1_body,
        grid=(NP // _BNP,),
        in_specs=[
            pl.BlockSpec((_BNP, 512), lambda i: (i, 0)),
            pl.BlockSpec((_BNP, 512), lambda i: (i, 0)),
            pl.BlockSpec((_BNP, 512), lambda i: (i, 0)),
            pl.BlockSpec((_BNP, 512), lambda i: (i, 0)),
            pl.BlockSpec((512, 256), lambda i: (0, 0)),
            pl.BlockSpec((512, 256), lambda i: (0, 0)),
            pl.BlockSpec((512, 256), lambda i: (0, 0)),
        ],
        out_specs=[
            pl.BlockSpec((_BNP, 512), lambda i: (i, 0)),
            pl.BlockSpec((_BNP, 512), lambda i: (i, 0)),
        ],
        out_shape=[
            jax.ShapeDtypeStruct((NP, 512), jnp.float32),
            jax.ShapeDtypeStruct((NP, 512), jnp.float32),
        ],
    )(x1a, x1b, ya, yb, bd02, bd1, bd2)


def _assemble_body(final, pa_ref, y2_ref, hx_ref, r_ref, bl_ref,
                   wl_ref, gb_ref, alpha_ref, pw_ref, *out_refs):
    pa = pa_ref[...]
    y2 = y2_ref[...]
    rb = r_ref[...]
    bl = bl_ref[...]
    wl = wl_ref[...]
    gb = gb_ref[...]
    outs = []
    projs = []
    for b in range(BS):
        conv_pre = pa[:, b * 64:(b + 1) * 64] + 2.0 * y2[:, b * 64:(b + 1) * 64] + gb
        conv = jnp.where(conv_pre >= 0, conv_pre, 0.01 * conv_pre)
        att = jnp.zeros_like(conv)
        for k in range(K):
            a_bk = alpha_ref[b, k]
            att = att + a_bk * (hx_ref[b, k] + rb[k])
        ob = jnp.dot(conv, wl, preferred_element_type=jnp.float32) + bl + att
        outs.append(ob)
        if final:
            projs.append(jnp.sum(ob * pw_ref[...], axis=1))
    out_refs[0][...] = jnp.stack(outs, axis=0)[:, None]
    if final:
        out_refs[1][...] = jnp.stack(projs, axis=0)[None]


def _assemble(final, pa, y2, hx_l, r_l, b_l, w_l, gb, alpha, pw):
    out_shape = [jax.ShapeDtypeStruct((BS, N // _BN, _BN, 64), jnp.float32)]
    out_specs = [pl.BlockSpec((BS, 1, _BN, 64), lambda i: (0, i, 0, 0))]
    if final:
        out_shape.append(jax.ShapeDtypeStruct((N // _BN, BS, _BN), jnp.float32))
        out_specs.append(pl.BlockSpec((1, BS, _BN), lambda i: (i, 0, 0)))
    res = pl.pallas_call(
        functools.partial(_assemble_body, final),
        grid=(N // _BN,),
        in_specs=[
            pl.BlockSpec((_BN, 512), lambda i: (i, 0)),
            pl.BlockSpec((_BN, 512), lambda i: (i, 0)),
            pl.BlockSpec((BS, K, _BN, 64), lambda i: (0, 0, i, 0)),
            pl.BlockSpec((K, _BN, 64), lambda i: (0, i, 0)),
            pl.BlockSpec((_BN, 64), lambda i: (i, 0)),
            pl.BlockSpec((64, 64), lambda i: (0, 0)),
            pl.BlockSpec((1, 64), lambda i: (0, 0)),
            pl.BlockSpec(memory_space=pltpu.SMEM),
            pl.BlockSpec((1, 64), lambda i: (0, 0)),
        ],
        out_specs=out_specs,
        out_shape=out_shape,
    )(pa, y2, hx_l, r_l, b_l, w_l, gb, alpha, pw)
    out_std = res[0].reshape(BS, N, 64)
    if final:
        return out_std, res[1].transpose(1, 0, 2).reshape(BS, N)
    return out_std, None


def _block_diag(w, isz_p, nb):
    isz = w.shape[0]
    wp = jnp.zeros((isz_p, 64), jnp.float32).at[:isz].set(w)
    out = jnp.zeros((nb * isz_p, nb * 64), jnp.float32)
    for b in range(nb):
        out = out.at[b * isz_p:(b + 1) * isz_p, b * 64:(b + 1) * 64].set(wp)
    return out


def kernel(inputs, hx_k, sup_rows, sup_cols, sup_vals, params):
    p = params

    alphas = []
    for l in range(2):
        aw = p['att_w_%d' % l].reshape(N, 64)
        s, c = _scores(hx_k[l], p['R_%d' % l], aw)
        alphas.append(jax.nn.softmax(s + c[None], axis=1))

    ws = []
    for l in range(2):
        w = p['gconv_w_%d' % l]
        ws.append((w[0::3], w[1::3], w[2::3]))

    order = jnp.argsort(sup_rows)
    rs = jnp.concatenate([jnp.take(sup_rows, order),
                          jnp.zeros((WGMAX,), jnp.int32)])
    cs = jnp.concatenate([jnp.take(sup_cols, order),
                          jnp.zeros((WGMAX,), jnp.int32)])
    vs = jnp.concatenate([jnp.take(sup_vals, order),
                          jnp.zeros((WGMAX,), jnp.float32)])
    wst = jnp.searchsorted(rs[:E], jnp.arange(NWIN + 1) * WROWS).astype(jnp.int32)
    wst = jnp.concatenate([wst, jnp.full((WSL - NWIN - 1,), E, jnp.int32)])

    xs0 = jnp.concatenate(
        [inputs.reshape(BS, N, 1), hx_k[0, :, K - 1]], axis=2)
    x0p = jnp.zeros((NP, BS, 80), jnp.float32)
    x0p = x0p.at[:N, :, :65].set(xs0.transpose(1, 0, 2)).reshape(NP, 640)

    w0, w1, w2 = ws[0]
    bd02_0 = _block_diag(w0 - w2, 80, 8)
    bd1_0 = _block_diag(w1, 80, 8)
    bd2_0 = _block_diag(w2, 80, 8)

    y1_0 = _spmm_640(x0p, rs, cs, vs, wst)
    u0, pa0 = _proj0(x0p, y1_0, bd02_0, bd1_0, bd2_0)
    y2_0 = _spmm_512(u0, rs, cs, vs, wst)

    alpha0 = jnp.zeros((8, 128), jnp.float32).at[:, :4].set(alphas[0])
    out0, _ = _assemble(
        False, pa0, y2_0, hx_k[0], p['R_0'], p['b_0'], p['W_0'],
        p['gconv_b_0'].reshape(1, 64), alpha0, jnp.zeros((1, 64), jnp.float32))

    xs1 = jnp.concatenate([out0, hx_k[1, :, K - 1]], axis=2)
    x1p = jnp.zeros((NP, BS, 128), jnp.float32)
    x1p = x1p.at[:N].set(xs1.transpose(1, 0, 2)).reshape(NP, 1024)
    x1a, x1b = x1p[:, :512], x1p[:, 512:]

    w0, w1, w2 = ws[1]
    bd02_1 = _block_diag(w0 - w2, 128, 4)
    bd1_1 = _block_diag(w1, 128, 4)
    bd2_1 = _block_diag(w2, 128, 4)

    y1_1a = _spmm_512(x1a, rs, cs, vs, wst)
    y1_1b = _spmm_512(x1b, rs, cs, vs, wst)
    u1, pa1 = _proj1(x1a, x1b, y1_1a, y1_1b, bd02_1, bd1_1, bd2_1)
    y2_1 = _spmm_512(u1, rs, cs, vs, wst)

    alpha1 = jnp.zeros((8, 128), jnp.float32).at[:, :4].set(alphas[1])
    out1, proj = _assemble(
        True, pa1, y2_1, hx_k[1], p['R_1'], p['b_1'], p['W_1'],
        p['gconv_b_1'].reshape(1, 64), alpha1,
        p['proj_w'].reshape(1, 64) + jnp.zeros((1, 64), jnp.float32))

    proj = proj + p['proj_b'][0]

    hx_out = jnp.stack([
        jnp.concatenate([hx_k[0, :, 1:], out0[:, None]], axis=1),
        jnp.concatenate([hx_k[1, :, 1:], out1[:, None]], axis=1),
    ])
    return proj, hx_out


# speedup vs baseline: 1.6072x; 1.6072x over previous
import functools

import jax
import jax.numpy as jnp
from jax import lax
from jax.experimental import pallas as pl
from jax.experimental.pallas import tpu as pltpu
from jax.experimental.pallas import tpu_sc as plsc

N = 10000
NP = 10240
E = 160000
D = 64
K = 4
BS = 8
NT = 16
NCORE = 2
ET = E // NT
C = 4
RC = NP // C
RT = RC // NT
G = 48


WROWS = 64
NWIN = NP // WROWS
WPT = NWIN // (NT * NCORE)
WGMAX = 8192
G = 16
EP = E + WGMAX
WSL = 176


def _vgather(x, idx):
    dn = lax.GatherDimensionNumbers(offset_dims=(), collapsed_slice_dims=(0,),
                                    start_index_map=(0,))
    return lax.gather(x, idx[:, None], dn, (1,),
                      mode=lax.GatherScatterMode.PROMISE_IN_BOUNDS)


def _make_spmm(W):
    WV = W // 16
    mesh = plsc.VectorSubcoreMesh(core_axis_name="c", subcore_axis_name="s",
                                  num_cores=NCORE, num_subcores=NT)

    @functools.partial(
        pl.kernel,
        out_type=jax.ShapeDtypeStruct((NP, W), jnp.float32),
        mesh=mesh,
        scratch_types=[
            pltpu.VMEM((WGMAX,), jnp.int32),
            pltpu.VMEM((WGMAX,), jnp.int32),
            pltpu.VMEM((WGMAX,), jnp.float32),
            pltpu.VMEM((WSL,), jnp.int32),
            pltpu.VMEM((WROWS + 8, W), jnp.float32),
            pltpu.VMEM((2, G, W), jnp.float32),
            pltpu.VMEM((2, G), jnp.int32),
            pltpu.VMEM((32,), jnp.int32),
            pltpu.SemaphoreType.DMA,
        ],
    )
    def spmm(x_hbm, rows_hbm, cols_hbm, vals_hbm, ws_hbm, y_hbm,
             rows_v, cols_v, vals_v, wsv, acc, gbuf, gidx, dbuf, gsem):
        cid = lax.axis_index("c")
        sid = lax.axis_index("s")
        wid = cid * NT + sid
        w0 = wid * WPT

        pltpu.sync_copy(ws_hbm, wsv)
        vstart = wsv[pl.ds(w0, 16)]
        e_start = vstart[0]
        e_end = vstart[WPT]
        astart = (e_start // 8) * 8
        e_end = jnp.minimum(e_end, astart + WGMAX)
        pltpu.sync_copy(rows_hbm.at[pl.ds(astart, WGMAX)], rows_v)
        pltpu.sync_copy(cols_hbm.at[pl.ds(astart, WGMAX)], cols_v)
        pltpu.sync_copy(vals_hbm.at[pl.ds(astart, WGMAX)], vals_v)

        zf = jnp.zeros((16,), jnp.float32)
        zi = jnp.zeros((16,), jnp.int32)
        iota = lax.iota(jnp.int32, 16)
        dbuf[pl.ds(16, 16)] = zi

        def zacc(r, _):
            for w in range(WV):
                acc[r, pl.ds(w * 16, 16)] = zf
            return 0
        lax.fori_loop(0, WROWS + 1, zacc, 0)

        for j in range(WPT):
            wlo = (w0 + j) * WROWS
            es = jnp.minimum(vstart[j], e_end)
            ee = jnp.minimum(vstart[j + 1], e_end)
            cnt = ee - es
            base0 = es - astart
            nb = (cnt + (G - 1)) // G

            def fill_gidx(k, buf):
                cc = cols_v[pl.ds(base0 + k * G, 16)]
                keep = (jnp.full((16,), k * G, jnp.int32) + iota) < jnp.full(
                    (16,), cnt, jnp.int32)
                gidx[buf, pl.ds(0, 16)] = jnp.where(keep, cc, zi)
                pltpu.make_async_copy(
                    x_hbm.at[gidx.at[buf]], gbuf.at[buf], gsem).start()

            @pl.when(nb > 0)
            def _():
                fill_gidx(0, 0)

            def blk(k, carry):
                buf = lax.rem(k, 2)
                pltpu.make_async_copy(
                    x_hbm.at[gidx.at[buf]], gbuf.at[buf], gsem).wait()
                @pl.when(k + 1 < nb)
                def _():
                    fill_gidx(k + 1, 1 - buf)
                rv = rows_v[pl.ds(base0 + k * G, 16)]
                vv = vals_v[pl.ds(base0 + k * G, 16)]
                keep = (jnp.full((16,), k * G, jnp.int32) + iota) < jnp.full(
                    (16,), cnt, jnp.int32)
                dstv = jnp.where(keep, rv - jnp.full((16,), wlo, jnp.int32),
                                 jnp.full((16,), WROWS, jnp.int32))
                vk = jnp.where(keep, vv, zf)
                dbuf[pl.ds(0, 16)] = dstv

                if WV <= 32:
                    accs = [zf] * WV
                    prev_d = jnp.int32(WROWS)
                    for l in range(G):
                        d = dbuf[pl.ds(l, 16)][0]
                        bv = _vgather(vk, jnp.full((16,), l, jnp.int32))
                        same = d == prev_d
                        @pl.when(jnp.logical_not(same))
                        def _(prev_d=prev_d, accs=tuple(accs)):
                            for w in range(WV):
                                acc[prev_d, pl.ds(w * 16, 16)] = (
                                    acc[prev_d, pl.ds(w * 16, 16)] + accs[w])
                        keepf = jnp.full((16,), jnp.where(same, 1.0, 0.0),
                                         jnp.float32)
                        accs = [gbuf[buf, l, pl.ds(w * 16, 16)] * bv
                                + accs[w] * keepf
                                for w in range(WV)]
                        prev_d = d
                    for w in range(WV):
                        acc[prev_d, pl.ds(w * 16, 16)] = (
                            acc[prev_d, pl.ds(w * 16, 16)] + accs[w])
                else:
                    def lane4(q, _):
                        for li in range(4):
                            l = q * 4 + li
                            d = dbuf[pl.ds(l, 16)][0]
                            bv = _vgather(vk, jnp.full((16,), l, jnp.int32))
                            for w in range(WV):
                                acc[d, pl.ds(w * 16, 16)] = (
                                    acc[d, pl.ds(w * 16, 16)]
                                    + gbuf[buf, l, pl.ds(w * 16, 16)] * bv)
                        return 0
                    lax.fori_loop(0, G // 4, lane4, 0)
                return 0
            lax.fori_loop(0, nb, blk, 0)

            pltpu.sync_copy(acc.at[pl.ds(0, WROWS)],
                            y_hbm.at[pl.ds(wlo, WROWS)])
            lax.fori_loop(0, WROWS + 1, zacc, 0)

    return spmm


_spmm_640 = _make_spmm(640)
_spmm_512 = _make_spmm(512)


_BN = 400
_BNP = 1024


def _scores_body(hx_ref, r_ref, aw_ref, s_ref, c_ref):
    i = pl.program_id(0)
    awb = aw_ref[...]
    ps = jnp.sum(hx_ref[...] * awb[None], axis=1)
    pc = jnp.sum(r_ref[...] * awb[None], axis=1)

    @pl.when(i == 0)
    def _():
        s_ref[...] = jnp.zeros_like(s_ref)
        c_ref[...] = jnp.zeros_like(c_ref)
    s_ref[...] += ps
    c_ref[...] += pc


def _scores(hx_l, r_l, aw_l):
    hx2 = hx_l.reshape(32, N, 64)
    s_part, c_part = pl.pallas_call(
        _scores_body,
        grid=(N // _BN,),
        in_specs=[
            pl.BlockSpec((32, _BN, 64), lambda i: (0, i, 0)),
            pl.BlockSpec((4, _BN, 64), lambda i: (0, i, 0)),
            pl.BlockSpec((_BN, 64), lambda i: (i, 0)),
        ],
        out_specs=[
            pl.BlockSpec((32, 64), lambda i: (0, 0)),
            pl.BlockSpec((4, 64), lambda i: (0, 0)),
        ],
        out_shape=[
            jax.ShapeDtypeStruct((32, 64), jnp.float32),
            jax.ShapeDtypeStruct((4, 64), jnp.float32),
        ],
    )(hx2, r_l, aw_l)
    return s_part.sum(axis=1).reshape(8, 4), c_part.sum(axis=1)


def _proj0_body(x0_ref, y1_ref, bd02_ref, bd1_ref, bd2_ref, u_ref, pa_ref):
    y1 = y1_ref[...]
    u_ref[...] = jnp.dot(y1, bd2_ref[...], preferred_element_type=jnp.float32)
    pa_ref[...] = (
        jnp.dot(x0_ref[...], bd02_ref[...], preferred_element_type=jnp.float32)
        + jnp.dot(y1, bd1_ref[...], preferred_element_type=jnp.float32))


def _proj0(x0p, y1, bd02, bd1, bd2):
    return pl.pallas_call(
        _proj0_body,
        grid=(NP // _BNP,),
        in_specs=[
            pl.BlockSpec((_BNP, 640), lambda i: (i, 0)),
            pl.BlockSpec((_BNP, 640), lambda i: (i, 0)),
            pl.BlockSpec((640, 512), lambda i: (0, 0)),
            pl.BlockSpec((640, 512), lambda i: (0, 0)),
            pl.BlockSpec((640, 512), lambda i: (0, 0)),
        ],
        out_specs=[
            pl.BlockSpec((_BNP, 512), lambda i: (i, 0)),
            pl.BlockSpec((_BNP, 512), lambda i: (i, 0)),
        ],
        out_shape=[
            jax.ShapeDtypeStruct((NP, 512), jnp.float32),
            jax.ShapeDtypeStruct((NP, 512), jnp.float32),
        ],
    )(x0p, y1, bd02, bd1, bd2)


def _proj1_body(x0a_ref, x0b_ref, ya_ref, yb_ref,
                bd02_ref, bd1_ref, bd2_ref, u_ref, pa_ref):
    bd02, bd1, bd2 = bd02_ref[...], bd1_ref[...], bd2_ref[...]
    ya = ya_ref[...]
    yb = yb_ref[...]
    ua = jnp.dot(ya, bd2, preferred_element_type=jnp.float32)
    ub = jnp.dot(yb, bd2, preferred_element_type=jnp.float32)
    u_ref[...] = jnp.concatenate([ua, ub], axis=1)
    pa = (jnp.dot(x0a_ref[...], bd02, preferred_element_type=jnp.float32)
          + jnp.dot(ya, bd1, preferred_element_type=jnp.float32))
    pb = (jnp.dot(x0b_ref[...], bd02, preferred_element_type=jnp.float32)
          + jnp.dot(yb, bd1, preferred_element_type=jnp.float32))
    pa_ref[...] = jnp.concatenate([pa, pb], axis=1)


def _proj1(x1a, x1b, ya, yb, bd02, bd1, bd2):
    return pl.pallas_call(
        _proj1_body,
        grid=(NP // _BNP,),
        in_specs=[
            pl.BlockSpec((_BNP, 512), lambda i: (i, 0)),
            pl.BlockSpec((_BNP, 512), lambda i: (i, 0)),
            pl.BlockSpec((_BNP, 512), lambda i: (i, 0)),
            pl.BlockSpec((_BNP, 512), lambda i: (i, 0)),
            pl.BlockSpec((512, 256), lambda i: (0, 0)),
            pl.BlockSpec((512, 256), lambda i: (0, 0)),
            pl.BlockSpec((512, 256), lambda i: (0, 0)),
        ],
        out_specs=[
            pl.BlockSpec((_BNP, 512), lambda i: (i, 0)),
            pl.BlockSpec((_BNP, 512), lambda i: (i, 0)),
        ],
        out_shape=[
            jax.ShapeDtypeStruct((NP, 512), jnp.float32),
            jax.ShapeDtypeStruct((NP, 512), jnp.float32),
        ],
    )(x1a, x1b, ya, yb, bd02, bd1, bd2)


def _assemble_body(final, pa_ref, y2_ref, hx_ref, r_ref, bl_ref,
                   wl_ref, gb_ref, alpha_ref, pw_ref, *out_refs):
    pa = pa_ref[...]
    y2 = y2_ref[...]
    rb = r_ref[...]
    bl = bl_ref[...]
    wl = wl_ref[...]
    gb = gb_ref[...]
    outs = []
    projs = []
    for b in range(BS):
        conv_pre = pa[:, b * 64:(b + 1) * 64] + 2.0 * y2[:, b * 64:(b + 1) * 64] + gb
        conv = jnp.where(conv_pre >= 0, conv_pre, 0.01 * conv_pre)
        att = jnp.zeros_like(conv)
        for k in range(K):
            a_bk = alpha_ref[b, k]
            att = att + a_bk * (hx_ref[b, k] + rb[k])
        ob = jnp.dot(conv, wl, preferred_element_type=jnp.float32) + bl + att
        outs.append(ob)
        if final:
            projs.append(jnp.sum(ob * pw_ref[...], axis=1))
    out_refs[0][...] = jnp.stack(outs, axis=0)[:, None]
    if final:
        out_refs[1][...] = jnp.stack(projs, axis=0)[None]


def _assemble(final, pa, y2, hx_l, r_l, b_l, w_l, gb, alpha, pw):
    out_shape = [jax.ShapeDtypeStruct((BS, N // _BN, _BN, 64), jnp.float32)]
    out_specs = [pl.BlockSpec((BS, 1, _BN, 64), lambda i: (0, i, 0, 0))]
    if final:
        out_shape.append(jax.ShapeDtypeStruct((N // _BN, BS, _BN), jnp.float32))
        out_specs.append(pl.BlockSpec((1, BS, _BN), lambda i: (i, 0, 0)))
    res = pl.pallas_call(
        functools.partial(_assemble_body, final),
        grid=(N // _BN,),
        in_specs=[
            pl.BlockSpec((_BN, 512), lambda i: (i, 0)),
            pl.BlockSpec((_BN, 512), lambda i: (i, 0)),
            pl.BlockSpec((BS, K, _BN, 64), lambda i: (0, 0, i, 0)),
            pl.BlockSpec((K, _BN, 64), lambda i: (0, i, 0)),
            pl.BlockSpec((_BN, 64), lambda i: (i, 0)),
            pl.BlockSpec((64, 64), lambda i: (0, 0)),
            pl.BlockSpec((1, 64), lambda i: (0, 0)),
            pl.BlockSpec(memory_space=pltpu.SMEM),
            pl.BlockSpec((1, 64), lambda i: (0, 0)),
        ],
        out_specs=out_specs,
        out_shape=out_shape,
    )(pa, y2, hx_l, r_l, b_l, w_l, gb, alpha, pw)
    out_std = res[0].reshape(BS, N, 64)
    if final:
        return out_std, res[1].transpose(1, 0, 2).reshape(BS, N)
    return out_std, None


def _block_diag(w, isz_p, nb):
    isz = w.shape[0]
    wp = jnp.zeros((isz_p, 64), jnp.float32).at[:isz].set(w)
    out = jnp.zeros((nb * isz_p, nb * 64), jnp.float32)
    for b in range(nb):
        out = out.at[b * isz_p:(b + 1) * isz_p, b * 64:(b + 1) * 64].set(wp)
    return out


def kernel(inputs, hx_k, sup_rows, sup_cols, sup_vals, params):
    p = params

    alphas = []
    for l in range(2):
        aw = p['att_w_%d' % l].reshape(N, 64)
        s, c = _scores(hx_k[l], p['R_%d' % l], aw)
        alphas.append(jax.nn.softmax(s + c[None], axis=1))

    ws = []
    for l in range(2):
        w = p['gconv_w_%d' % l]
        ws.append((w[0::3], w[1::3], w[2::3]))

    order = jnp.argsort(sup_rows)
    rs = jnp.concatenate([jnp.take(sup_rows, order),
                          jnp.zeros((WGMAX,), jnp.int32)])
    cs = jnp.concatenate([jnp.take(sup_cols, order),
                          jnp.zeros((WGMAX,), jnp.int32)])
    vs = jnp.concatenate([jnp.take(sup_vals, order),
                          jnp.zeros((WGMAX,), jnp.float32)])
    wst = jnp.searchsorted(rs[:E], jnp.arange(NWIN + 1) * WROWS).astype(jnp.int32)
    wst = jnp.concatenate([wst, jnp.full((WSL - NWIN - 1,), E, jnp.int32)])

    xs0 = jnp.concatenate(
        [inputs.reshape(BS, N, 1), hx_k[0, :, K - 1]], axis=2)
    x0p = jnp.zeros((NP, BS, 80), jnp.float32)
    x0p = x0p.at[:N, :, :65].set(xs0.transpose(1, 0, 2)).reshape(NP, 640)

    w0, w1, w2 = ws[0]
    bd02_0 = _block_diag(w0 - w2, 80, 8)
    bd1_0 = _block_diag(w1, 80, 8)
    bd2_0 = _block_diag(w2, 80, 8)

    y1_0 = _spmm_640(x0p, rs, cs, vs, wst)
    u0, pa0 = _proj0(x0p, y1_0, bd02_0, bd1_0, bd2_0)
    y2_0 = _spmm_512(u0, rs, cs, vs, wst)

    alpha0 = jnp.zeros((8, 128), jnp.float32).at[:, :4].set(alphas[0])
    out0, _ = _assemble(
        False, pa0, y2_0, hx_k[0], p['R_0'], p['b_0'], p['W_0'],
        p['gconv_b_0'].reshape(1, 64), alpha0, jnp.zeros((1, 64), jnp.float32))

    xs1 = jnp.concatenate([out0, hx_k[1, :, K - 1]], axis=2)
    x1p = jnp.zeros((NP, BS, 128), jnp.float32)
    x1p = x1p.at[:N].set(xs1.transpose(1, 0, 2)).reshape(NP, 1024)
    x1a, x1b = x1p[:, :512], x1p[:, 512:]

    w0, w1, w2 = ws[1]
    bd02_1 = _block_diag(w0 - w2, 128, 4)
    bd1_1 = _block_diag(w1, 128, 4)
    bd2_1 = _block_diag(w2, 128, 4)

    y1_1a = _spmm_512(x1a, rs, cs, vs, wst)
    y1_1b = _spmm_512(x1b, rs, cs, vs, wst)
    u1, pa1 = _proj1(x1a, x1b, y1_1a, y1_1b, bd02_1, bd1_1, bd2_1)
    y2_1 = _spmm_512(u1, rs, cs, vs, wst)

    alpha1 = jnp.zeros((8, 128), jnp.float32).at[:, :4].set(alphas[1])
    out1, proj = _assemble(
        True, pa1, y2_1, hx_k[1], p['R_1'], p['b_1'], p['W_1'],
        p['gconv_b_1'].reshape(1, 64), alpha1,
        p['proj_w'].reshape(1, 64) + jnp.zeros((1, 64), jnp.float32))

    proj = proj + p['proj_b'][0]

    hx_out = jnp.stack([
        jnp.concatenate([hx_k[0, :, 1:], out0[:, None]], axis=1),
        jnp.concatenate([hx_k[1, :, 1:], out1[:, None]], axis=1),
    ])
    return proj, hx_out


# speedup vs baseline: 2.0667x; 1.2859x over previous
import functools

import jax
import jax.numpy as jnp
from jax import lax
from jax.experimental import pallas as pl
from jax.experimental.pallas import tpu as pltpu
from jax.experimental.pallas import tpu_sc as plsc

N = 10000
NP = 10240
E = 160000
D = 64
K = 4
BS = 8
NT = 16
NCORE = 2
ET = E // NT
C = 4
RC = NP // C
RT = RC // NT
G = 48


WROWS = 64
NWIN = NP // WROWS
WPT = NWIN // (NT * NCORE)
WGMAX = 8192
G = 16
EP = E + WGMAX
WSL = 176


def _vgather(x, idx):
    dn = lax.GatherDimensionNumbers(offset_dims=(), collapsed_slice_dims=(0,),
                                    start_index_map=(0,))
    return lax.gather(x, idx[:, None], dn, (1,),
                      mode=lax.GatherScatterMode.PROMISE_IN_BOUNDS)


def _make_spmm(W):
    WV = W // 16
    mesh = plsc.VectorSubcoreMesh(core_axis_name="c", subcore_axis_name="s",
                                  num_cores=NCORE, num_subcores=NT)

    @functools.partial(
        pl.kernel,
        out_type=jax.ShapeDtypeStruct((NP, W), jnp.float32),
        mesh=mesh,
        scratch_types=[
            pltpu.VMEM((WGMAX,), jnp.int32),
            pltpu.VMEM((WGMAX,), jnp.int32),
            pltpu.VMEM((WGMAX,), jnp.float32),
            pltpu.VMEM((WSL,), jnp.int32),
            pltpu.VMEM((WROWS + 8, W), jnp.float32),
            pltpu.VMEM((2, G, W), jnp.float32),
            pltpu.VMEM((2, G), jnp.int32),
            pltpu.VMEM((32,), jnp.int32),
            pltpu.SemaphoreType.DMA,
        ],
    )
    def spmm(x_hbm, rows_hbm, cols_hbm, vals_hbm, ws_hbm, y_hbm,
             rows_v, cols_v, vals_v, wsv, acc, gbuf, gidx, dbuf, gsem):
        cid = lax.axis_index("c")
        sid = lax.axis_index("s")
        wid = cid * NT + sid
        w0 = wid * WPT

        pltpu.sync_copy(ws_hbm, wsv)
        vstart = wsv[pl.ds(w0, 16)]
        e_start = vstart[0]
        e_end = vstart[WPT]
        astart = (e_start // 8) * 8
        e_end = jnp.minimum(e_end, astart + WGMAX)
        pltpu.sync_copy(rows_hbm.at[pl.ds(astart, WGMAX)], rows_v)
        pltpu.sync_copy(cols_hbm.at[pl.ds(astart, WGMAX)], cols_v)
        pltpu.sync_copy(vals_hbm.at[pl.ds(astart, WGMAX)], vals_v)

        zf = jnp.zeros((16,), jnp.float32)
        zi = jnp.zeros((16,), jnp.int32)
        iota = lax.iota(jnp.int32, 16)
        dbuf[pl.ds(16, 16)] = zi

        def zacc(r, _):
            for w in range(WV):
                acc[r, pl.ds(w * 16, 16)] = zf
            return 0
        lax.fori_loop(0, WROWS + 1, zacc, 0)

        for j in range(WPT):
            wlo = (w0 + j) * WROWS
            es = jnp.minimum(vstart[j], e_end)
            ee = jnp.minimum(vstart[j + 1], e_end)
            cnt = ee - es
            base0 = es - astart
            nb = (cnt + (G - 1)) // G

            def fill_gidx(k, buf):
                cc = cols_v[pl.ds(base0 + k * G, 16)]
                keep = (jnp.full((16,), k * G, jnp.int32) + iota) < jnp.full(
                    (16,), cnt, jnp.int32)
                gidx[buf, pl.ds(0, 16)] = jnp.where(keep, cc, zi)
                pltpu.make_async_copy(
                    x_hbm.at[gidx.at[buf]], gbuf.at[buf], gsem).start()

            @pl.when(nb > 0)
            def _():
                fill_gidx(0, 0)

            def blk(k, carry):
                buf = lax.rem(k, 2)
                pltpu.make_async_copy(
                    x_hbm.at[gidx.at[buf]], gbuf.at[buf], gsem).wait()
                @pl.when(k + 1 < nb)
                def _():
                    fill_gidx(k + 1, 1 - buf)
                rv = rows_v[pl.ds(base0 + k * G, 16)]
                vv = vals_v[pl.ds(base0 + k * G, 16)]
                keep = (jnp.full((16,), k * G, jnp.int32) + iota) < jnp.full(
                    (16,), cnt, jnp.int32)
                dstv = jnp.where(keep, rv - jnp.full((16,), wlo, jnp.int32),
                                 jnp.full((16,), WROWS, jnp.int32))
                vk = jnp.where(keep, vv, zf)
                dbuf[pl.ds(0, 16)] = dstv

                LST = 16 if WV <= 32 else 8

                def lgroup(q, _):
                    accs = [zf] * WV
                    prev_d = jnp.int32(WROWS)
                    for li in range(LST):
                        l = q * LST + li
                        d = dbuf[pl.ds(l, 16)][0]
                        bv = _vgather(vk, jnp.full((16,), l, jnp.int32))
                        same = d == prev_d
                        @pl.when(jnp.logical_not(same))
                        def _(prev_d=prev_d, accs=tuple(accs)):
                            for w in range(WV):
                                acc[prev_d, pl.ds(w * 16, 16)] = (
                                    acc[prev_d, pl.ds(w * 16, 16)] + accs[w])
                        keepf = jnp.full((16,), jnp.where(same, 1.0, 0.0),
                                         jnp.float32)
                        accs = [gbuf[buf, l, pl.ds(w * 16, 16)] * bv
                                + accs[w] * keepf
                                for w in range(WV)]
                        prev_d = d
                    for w in range(WV):
                        acc[prev_d, pl.ds(w * 16, 16)] = (
                            acc[prev_d, pl.ds(w * 16, 16)] + accs[w])
                    return 0
                lax.fori_loop(0, G // LST, lgroup, 0)
                return 0
            lax.fori_loop(0, nb, blk, 0)

            pltpu.sync_copy(acc.at[pl.ds(0, WROWS)],
                            y_hbm.at[pl.ds(wlo, WROWS)])
            lax.fori_loop(0, WROWS + 1, zacc, 0)

    return spmm


_spmm_640 = _make_spmm(640)
_spmm_512 = _make_spmm(512)


_BN = 400
_BNP = 1024


def _scores_body(hx_ref, r_ref, aw_ref, s_ref, c_ref):
    i = pl.program_id(0)
    awb = aw_ref[...]
    ps = jnp.sum(hx_ref[...] * awb[None], axis=1)
    pc = jnp.sum(r_ref[...] * awb[None], axis=1)

    @pl.when(i == 0)
    def _():
        s_ref[...] = jnp.zeros_like(s_ref)
        c_ref[...] = jnp.zeros_like(c_ref)
    s_ref[...] += ps
    c_ref[...] += pc


def _scores(hx_l, r_l, aw_l):
    hx2 = hx_l.reshape(32, N, 64)
    s_part, c_part = pl.pallas_call(
        _scores_body,
        grid=(N // _BN,),
        in_specs=[
            pl.BlockSpec((32, _BN, 64), lambda i: (0, i, 0)),
            pl.BlockSpec((4, _BN, 64), lambda i: (0, i, 0)),
            pl.BlockSpec((_BN, 64), lambda i: (i, 0)),
        ],
        out_specs=[
            pl.BlockSpec((32, 64), lambda i: (0, 0)),
            pl.BlockSpec((4, 64), lambda i: (0, 0)),
        ],
        out_shape=[
            jax.ShapeDtypeStruct((32, 64), jnp.float32),
            jax.ShapeDtypeStruct((4, 64), jnp.float32),
        ],
    )(hx2, r_l, aw_l)
    return s_part.sum(axis=1).reshape(8, 4), c_part.sum(axis=1)


def _proj0_body(x0_ref, y1_ref, bd02_ref, bd1_ref, bd2_ref, u_ref, pa_ref):
    y1 = y1_ref[...]
    u_ref[...] = jnp.dot(y1, bd2_ref[...], preferred_element_type=jnp.float32)
    pa_ref[...] = (
        jnp.dot(x0_ref[...], bd02_ref[...], preferred_element_type=jnp.float32)
        + jnp.dot(y1, bd1_ref[...], preferred_element_type=jnp.float32))


def _proj0(x0p, y1, bd02, bd1, bd2):
    return pl.pallas_call(
        _proj0_body,
        grid=(NP // _BNP,),
        in_specs=[
            pl.BlockSpec((_BNP, 640), lambda i: (i, 0)),
            pl.BlockSpec((_BNP, 640), lambda i: (i, 0)),
            pl.BlockSpec((640, 512), lambda i: (0, 0)),
            pl.BlockSpec((640, 512), lambda i: (0, 0)),
            pl.BlockSpec((640, 512), lambda i: (0, 0)),
        ],
        out_specs=[
            pl.BlockSpec((_BNP, 512), lambda i: (i, 0)),
            pl.BlockSpec((_BNP, 512), lambda i: (i, 0)),
        ],
        out_shape=[
            jax.ShapeDtypeStruct((NP, 512), jnp.float32),
            jax.ShapeDtypeStruct((NP, 512), jnp.float32),
        ],
    )(x0p, y1, bd02, bd1, bd2)


def _proj1_body(x0a_ref, x0b_ref, ya_ref, yb_ref,
                bd02_ref, bd1_ref, bd2_ref, u_ref, pa_ref):
    bd02, bd1, bd2 = bd02_ref[...], bd1_ref[...], bd2_ref[...]
    ya = ya_ref[...]
    yb = yb_ref[...]
    ua = jnp.dot(ya, bd2, preferred_element_type=jnp.float32)
    ub = jnp.dot(yb, bd2, preferred_element_type=jnp.float32)
    u_ref[...] = jnp.concatenate([ua, ub], axis=1)
    pa = (jnp.dot(x0a_ref[...], bd02, preferred_element_type=jnp.float32)
          + jnp.dot(ya, bd1, preferred_element_type=jnp.float32))
    pb = (jnp.dot(x0b_ref[...], bd02, preferred_element_type=jnp.float32)
          + jnp.dot(yb, bd1, preferred_element_type=jnp.float32))
    pa_ref[...] = jnp.concatenate([pa, pb], axis=1)


def _proj1(x1a, x1b, ya, yb, bd02, bd1, bd2):
    return pl.pallas_call(
        _proj1_body,
        grid=(NP // _BNP,),
        in_specs=[
            pl.BlockSpec((_BNP, 512), lambda i: (i, 0)),
            pl.BlockSpec((_BNP, 512), lambda i: (i, 0)),
            pl.BlockSpec((_BNP, 512), lambda i: (i, 0)),
            pl.BlockSpec((_BNP, 512), lambda i: (i, 0)),
            pl.BlockSpec((512, 256), lambda i: (0, 0)),
            pl.BlockSpec((512, 256), lambda i: (0, 0)),
            pl.BlockSpec((512, 256), lambda i: (0, 0)),
        ],
        out_specs=[
            pl.BlockSpec((_BNP, 512), lambda i: (i, 0)),
            pl.BlockSpec((_BNP, 512), lambda i: (i, 0)),
        ],
        out_shape=[
            jax.ShapeDtypeStruct((NP, 512), jnp.float32),
            jax.ShapeDtypeStruct((NP, 512), jnp.float32),
        ],
    )(x1a, x1b, ya, yb, bd02, bd1, bd2)


def _assemble_body(final, pa_ref, y2_ref, hx_ref, r_ref, bl_ref,
                   wl_ref, gb_ref, alpha_ref, pw_ref, *out_refs):
    pa = pa_ref[...]
    y2 = y2_ref[...]
    rb = r_ref[...]
    bl = bl_ref[...]
    wl = wl_ref[...]
    gb = gb_ref[...]
    outs = []
    projs = []
    for b in range(BS):
        conv_pre = pa[:, b * 64:(b + 1) * 64] + 2.0 * y2[:, b * 64:(b + 1) * 64] + gb
        conv = jnp.where(conv_pre >= 0, conv_pre, 0.01 * conv_pre)
        att = jnp.zeros_like(conv)
        for k in range(K):
            a_bk = alpha_ref[b, k]
            att = att + a_bk * (hx_ref[b, k] + rb[k])
        ob = jnp.dot(conv, wl, preferred_element_type=jnp.float32) + bl + att
        outs.append(ob)
        if final:
            projs.append(jnp.sum(ob * pw_ref[...], axis=1))
    out_refs[0][...] = jnp.stack(outs, axis=0)[:, None]
    if final:
        out_refs[1][...] = jnp.stack(projs, axis=0)[None]


def _assemble(final, pa, y2, hx_l, r_l, b_l, w_l, gb, alpha, pw):
    out_shape = [jax.ShapeDtypeStruct((BS, N // _BN, _BN, 64), jnp.float32)]
    out_specs = [pl.BlockSpec((BS, 1, _BN, 64), lambda i: (0, i, 0, 0))]
    if final:
        out_shape.append(jax.ShapeDtypeStruct((N // _BN, BS, _BN), jnp.float32))
        out_specs.append(pl.BlockSpec((1, BS, _BN), lambda i: (i, 0, 0)))
    res = pl.pallas_call(
        functools.partial(_assemble_body, final),
        grid=(N // _BN,),
        in_specs=[
            pl.BlockSpec((_BN, 512), lambda i: (i, 0)),
            pl.BlockSpec((_BN, 512), lambda i: (i, 0)),
            pl.BlockSpec((BS, K, _BN, 64), lambda i: (0, 0, i, 0)),
            pl.BlockSpec((K, _BN, 64), lambda i: (0, i, 0)),
            pl.BlockSpec((_BN, 64), lambda i: (i, 0)),
            pl.BlockSpec((64, 64), lambda i: (0, 0)),
            pl.BlockSpec((1, 64), lambda i: (0, 0)),
            pl.BlockSpec(memory_space=pltpu.SMEM),
            pl.BlockSpec((1, 64), lambda i: (0, 0)),
        ],
        out_specs=out_specs,
        out_shape=out_shape,
    )(pa, y2, hx_l, r_l, b_l, w_l, gb, alpha, pw)
    out_std = res[0].reshape(BS, N, 64)
    if final:
        return out_std, res[1].transpose(1, 0, 2).reshape(BS, N)
    return out_std, None


def _block_diag(w, isz_p, nb):
    isz = w.shape[0]
    wp = jnp.zeros((isz_p, 64), jnp.float32).at[:isz].set(w)
    out = jnp.zeros((nb * isz_p, nb * 64), jnp.float32)
    for b in range(nb):
        out = out.at[b * isz_p:(b + 1) * isz_p, b * 64:(b + 1) * 64].set(wp)
    return out


def kernel(inputs, hx_k, sup_rows, sup_cols, sup_vals, params):
    p = params

    alphas = []
    for l in range(2):
        aw = p['att_w_%d' % l].reshape(N, 64)
        s, c = _scores(hx_k[l], p['R_%d' % l], aw)
        alphas.append(jax.nn.softmax(s + c[None], axis=1))

    ws = []
    for l in range(2):
        w = p['gconv_w_%d' % l]
        ws.append((w[0::3], w[1::3], w[2::3]))

    order = jnp.argsort(sup_rows)
    rs = jnp.concatenate([jnp.take(sup_rows, order),
                          jnp.zeros((WGMAX,), jnp.int32)])
    cs = jnp.concatenate([jnp.take(sup_cols, order),
                          jnp.zeros((WGMAX,), jnp.int32)])
    vs = jnp.concatenate([jnp.take(sup_vals, order),
                          jnp.zeros((WGMAX,), jnp.float32)])
    wst = jnp.searchsorted(rs[:E], jnp.arange(NWIN + 1) * WROWS).astype(jnp.int32)
    wst = jnp.concatenate([wst, jnp.full((WSL - NWIN - 1,), E, jnp.int32)])

    xs0 = jnp.concatenate(
        [inputs.reshape(BS, N, 1), hx_k[0, :, K - 1]], axis=2)
    x0p = jnp.zeros((NP, BS, 80), jnp.float32)
    x0p = x0p.at[:N, :, :65].set(xs0.transpose(1, 0, 2)).reshape(NP, 640)

    w0, w1, w2 = ws[0]
    bd02_0 = _block_diag(w0 - w2, 80, 8)
    bd1_0 = _block_diag(w1, 80, 8)
    bd2_0 = _block_diag(w2, 80, 8)

    y1_0 = _spmm_640(x0p, rs, cs, vs, wst)
    u0, pa0 = _proj0(x0p, y1_0, bd02_0, bd1_0, bd2_0)
    y2_0 = _spmm_512(u0, rs, cs, vs, wst)

    alpha0 = jnp.zeros((8, 128), jnp.float32).at[:, :4].set(alphas[0])
    out0, _ = _assemble(
        False, pa0, y2_0, hx_k[0], p['R_0'], p['b_0'], p['W_0'],
        p['gconv_b_0'].reshape(1, 64), alpha0, jnp.zeros((1, 64), jnp.float32))

    xs1 = jnp.concatenate([out0, hx_k[1, :, K - 1]], axis=2)
    x1p = jnp.zeros((NP, BS, 128), jnp.float32)
    x1p = x1p.at[:N].set(xs1.transpose(1, 0, 2)).reshape(NP, 1024)
    x1a, x1b = x1p[:, :512], x1p[:, 512:]

    w0, w1, w2 = ws[1]
    bd02_1 = _block_diag(w0 - w2, 128, 4)
    bd1_1 = _block_diag(w1, 128, 4)
    bd2_1 = _block_diag(w2, 128, 4)

    y1_1a = _spmm_512(x1a, rs, cs, vs, wst)
    y1_1b = _spmm_512(x1b, rs, cs, vs, wst)
    u1, pa1 = _proj1(x1a, x1b, y1_1a, y1_1b, bd02_1, bd1_1, bd2_1)
    y2_1 = _spmm_512(u1, rs, cs, vs, wst)

    alpha1 = jnp.zeros((8, 128), jnp.float32).at[:, :4].set(alphas[1])
    out1, proj = _assemble(
        True, pa1, y2_1, hx_k[1], p['R_1'], p['b_1'], p['W_1'],
        p['gconv_b_1'].reshape(1, 64), alpha1,
        p['proj_w'].reshape(1, 64) + jnp.zeros((1, 64), jnp.float32))

    proj = proj + p['proj_b'][0]

    hx_out = jnp.stack([
        jnp.concatenate([hx_k[0, :, 1:], out0[:, None]], axis=1),
        jnp.concatenate([hx_k[1, :, 1:], out1[:, None]], axis=1),
    ])
    return proj, hx_out
